# Initial kernel scaffold; baseline (speedup 1.0000x reference)
import jax
import jax.numpy as jnp
from jax.experimental import pallas as pl


def kernel(new_indices, old_indices, old_cc, new_cc, vertex_positions):
    raise NotImplementedError("write your pallas kernel here")


# speedup vs baseline: 4.0717x; 4.0717x over previous
import functools

import jax
import jax.numpy as jnp
from jax import lax
from jax.experimental import pallas as pl
from jax.experimental.pallas import tpu as pltpu
from jax.experimental.pallas import tpu_sc as plsc

T_NEW = 8192
T_OLD = 32768
V = 10000
K = 5

NW = 32
LANES = 16
W_OLD = T_OLD // NW
W_NEW = T_NEW // NW

Q_TILE = 256
N_QT = T_NEW // Q_TILE
KCH = 4096
N_KCH = T_OLD // KCH

_PAIRS = ((0, 1), (0, 2), (0, 3), (1, 2), (1, 3), (2, 3))


def _prep_body(hoi0, hoi1, hoi2, hoi3, hoc0, hoc1, hoc2,
               hni0, hni1, hni2, hni3, hnc0, hnc1, hnc2, vp,
               kx, ky, kz, ksq, omel2, qx, qy, qz, qsq, nmel2,
               vp_v,
               oi0, oi1, oi2, oi3, oc0, oc1, oc2,
               ni0, ni1, ni2, ni3, nc0, nc1, nc2,
               ko0, ko1, ko2, ksq_v, om_v,
               qo0, qo1, qo2, qsq_v, nm_v):
    wid = lax.axis_index("s") * 2 + lax.axis_index("c")
    oi_v = [oi0, oi1, oi2, oi3]
    ni_v = [ni0, ni1, ni2, ni3]
    occ_v = [oc0, oc1, oc2]
    ncc_v = [nc0, nc1, nc2]
    ko_v = [ko0, ko1, ko2]
    qo_v = [qo0, qo1, qo2]

    hoi = [hoi0, hoi1, hoi2, hoi3]
    hni = [hni0, hni1, hni2, hni3]
    hoc = [hoc0, hoc1, hoc2]
    hnc = [hnc0, hnc1, hnc2]

    pltpu.sync_copy(vp, vp_v)
    for r in range(4):
        pltpu.sync_copy(hoi[r].at[pl.ds(wid * W_OLD, W_OLD)], oi_v[r])
        pltpu.sync_copy(hni[r].at[pl.ds(wid * W_NEW, W_NEW)], ni_v[r])
    for c in range(3):
        pltpu.sync_copy(hoc[c].at[pl.ds(wid * W_OLD, W_OLD)], occ_v[c])
        pltpu.sync_copy(hnc[c].at[pl.ds(wid * W_NEW, W_NEW)], ncc_v[c])

    def make_loop(idx_v, cc_v, out_v, sq_v, mel_v, width):
        def body(j, _):
            s = pl.ds(j * LANES, LANES)
            addrs = [idx_v[r][s] * 3 for r in range(4)]
            coords = [[plsc.load_gather(vp_v, [a + c]) for a in addrs]
                      for c in range(3)]
            pts = []
            for c in range(3):
                x0, x1, x2, x3 = coords[c]
                cent = ((x0 + x2) + (x1 + x3)) * 0.25
                pts.append(0.5 * (cent + cc_v[c][s]))
            px, py, pz = pts
            sq_v[s] = (px * px + pz * pz) + py * py
            mel = None
            for (a, b) in _PAIRS:
                dx = coords[0][a] - coords[0][b]
                dy = coords[1][a] - coords[1][b]
                dz = coords[2][a] - coords[2][b]
                e2 = (dx * dx + dy * dy) + dz * dz
                mel = e2 if mel is None else jnp.minimum(mel, e2)
            mel_v[s] = mel
            for c in range(3):
                out_v[c][s] = pts[c]
            return 0

        lax.fori_loop(0, width // LANES, body, 0)

    make_loop(oi_v, occ_v, ko_v, ksq_v, om_v, W_OLD)
    make_loop(ni_v, ncc_v, qo_v, qsq_v, nm_v, W_NEW)

    hko = [kx, ky, kz]
    hqo = [qx, qy, qz]
    for c in range(3):
        pltpu.sync_copy(ko_v[c], hko[c].at[pl.ds(wid * W_OLD, W_OLD)])
        pltpu.sync_copy(qo_v[c], hqo[c].at[pl.ds(wid * W_NEW, W_NEW)])
    pltpu.sync_copy(ksq_v, ksq.at[pl.ds(wid * W_OLD, W_OLD)])
    pltpu.sync_copy(om_v, omel2.at[pl.ds(wid * W_OLD, W_OLD)])
    pltpu.sync_copy(qsq_v, qsq.at[pl.ds(wid * W_NEW, W_NEW)])
    pltpu.sync_copy(nm_v, nmel2.at[pl.ds(wid * W_NEW, W_NEW)])


def _run_prep(oi_rows, oc_rows, ni_rows, nc_rows, vp):
    f = pl.kernel(
        _prep_body,
        out_type=(
            [jax.ShapeDtypeStruct((T_OLD,), jnp.float32)] * 5
            + [jax.ShapeDtypeStruct((T_NEW,), jnp.float32)] * 5
        ),
        mesh=plsc.VectorSubcoreMesh(core_axis_name="c", subcore_axis_name="s"),
        compiler_params=pltpu.CompilerParams(needs_layout_passes=False),
        scratch_types=(
            [pltpu.VMEM((V * 3,), jnp.float32)]
            + [pltpu.VMEM((W_OLD,), jnp.int32)] * 4
            + [pltpu.VMEM((W_OLD,), jnp.float32)] * 3
            + [pltpu.VMEM((W_NEW,), jnp.int32)] * 4
            + [pltpu.VMEM((W_NEW,), jnp.float32)] * 3
            + [pltpu.VMEM((W_OLD,), jnp.float32)] * 5
            + [pltpu.VMEM((W_NEW,), jnp.float32)] * 5
        ),
    )
    return f(*oi_rows, *oc_rows, *ni_rows, *nc_rows, vp.reshape(V * 3))


_BIG_I = 2 ** 30


def _knn_body(q8, k8, ksq, qsqb, od2, nd2,
              cands_o, wts_o, omel_o, nmel_o):
    qb = q8[...].astype(jnp.bfloat16)
    qsq = qsqb[...][:, 0:1]

    def extract5(d2, iot):
        vs, is_ = [], []
        for _ in range(K):
            m = jnp.min(d2, axis=1, keepdims=True)
            mi = jnp.min(jnp.where(d2 == m, iot, _BIG_I),
                         axis=1, keepdims=True)
            d2 = jnp.where(iot == mi, jnp.inf, d2)
            vs.append(m)
            is_.append(mi)
        return jnp.concatenate(vs, axis=1), jnp.concatenate(is_, axis=1)

    def chunk(c, carry):
        bv, bi = carry
        kblk = k8[:, pl.ds(c * KCH, KCH)].astype(jnp.bfloat16)
        qk = jnp.dot(qb, kblk, preferred_element_type=jnp.float32)
        a = qsq + ksq[:, pl.ds(c * KCH, KCH)]
        d2 = a - qk * 2.0
        iot = lax.broadcasted_iota(jnp.int32, (Q_TILE, KCH), 1) + c * KCH
        cv, ci = extract5(d2, iot)
        av = jnp.concatenate([bv, cv], axis=1)
        ai = jnp.concatenate([bi, ci], axis=1)
        nv, ni = extract5(av, ai)
        return nv, ni

    bv0 = jnp.full((Q_TILE, K), jnp.inf, jnp.float32)
    bi0 = jnp.full((Q_TILE, K), _BIG_I, jnp.int32)
    bv, bi = lax.fori_loop(0, N_KCH, chunk, (bv0, bi0))

    cands_o[...] = bi
    dist = jnp.sqrt(jnp.maximum(bv, 1e-12))
    z = -dist
    zm = jnp.max(z, axis=1, keepdims=True)
    e = jnp.exp(z - zm)
    wts_o[...] = e / jnp.sum(e, axis=1, keepdims=True)
    omel_o[...] = jnp.sqrt(od2[...] + 1e-12)
    nmel_o[...] = jnp.sqrt(nd2[...] + 1e-12)


def _run_knn(q8, k8, ksq2, qsqb, od2, nd2):
    return pl.pallas_call(
        _knn_body,
        grid=(N_QT,),
        in_specs=[
            pl.BlockSpec((Q_TILE, 8), lambda i: (i, 0)),
            pl.BlockSpec((8, T_OLD), lambda i: (0, 0)),
            pl.BlockSpec((1, T_OLD), lambda i: (0, 0)),
            pl.BlockSpec((Q_TILE, 128), lambda i: (i, 0)),
            pl.BlockSpec((1, W_OLD), lambda i: (0, i)),
            pl.BlockSpec((1, W_NEW), lambda i: (0, i)),
        ],
        out_specs=[
            pl.BlockSpec((Q_TILE, K), lambda i: (i, 0)),
            pl.BlockSpec((Q_TILE, K), lambda i: (i, 0)),
            pl.BlockSpec((1, W_OLD), lambda i: (0, i)),
            pl.BlockSpec((1, W_NEW), lambda i: (0, i)),
        ],
        out_shape=[
            jax.ShapeDtypeStruct((T_NEW, K), jnp.int32),
            jax.ShapeDtypeStruct((T_NEW, K), jnp.float32),
            jax.ShapeDtypeStruct((1, T_OLD), jnp.float32),
            jax.ShapeDtypeStruct((1, T_NEW), jnp.float32),
        ],
    )(q8, k8, ksq2, qsqb, od2, nd2)


W_D = T_NEW * K // NW


def _dens_body(omel, cands, nmexp, dens, mel_v, idx_v, nm_v, out_v):
    wid = lax.axis_index("s") * 2 + lax.axis_index("c")
    base = wid * W_D
    pltpu.sync_copy(omel, mel_v)
    pltpu.sync_copy(cands.at[pl.ds(base, W_D)], idx_v)
    pltpu.sync_copy(nmexp.at[pl.ds(base, W_D)], nm_v)

    def body(j, _):
        s = pl.ds(j * LANES, LANES)
        g = plsc.load_gather(mel_v, [idx_v[s]])
        out_v[s] = nm_v[s] / (g + 1e-8)
        return 0

    lax.fori_loop(0, W_D // LANES, body, 0)
    pltpu.sync_copy(out_v, dens.at[pl.ds(base, W_D)])


def _run_dens(omel, cands_flat, nmexp_flat):
    f = pl.kernel(
        _dens_body,
        out_type=[jax.ShapeDtypeStruct((T_NEW * K,), jnp.float32)],
        mesh=plsc.VectorSubcoreMesh(core_axis_name="c", subcore_axis_name="s"),
        compiler_params=pltpu.CompilerParams(needs_layout_passes=False),
        scratch_types=[
            pltpu.VMEM((T_OLD,), jnp.float32),
            pltpu.VMEM((W_D,), jnp.int32),
            pltpu.VMEM((W_D,), jnp.float32),
            pltpu.VMEM((W_D,), jnp.float32),
        ],
    )
    return f(omel, cands_flat, nmexp_flat)


def kernel(new_indices, old_indices, old_cc, new_cc, vertex_positions):
    oi_rows = [old_indices[:, r] for r in range(4)]
    ni_rows = [new_indices[:, r] for r in range(4)]
    oc_rows = [old_cc[:, c] for c in range(3)]
    nc_rows = [new_cc[:, c] for c in range(3)]

    kx, ky, kz, ksq, omel2, qx, qy, qz, qsq, nmel2 = _run_prep(
        oi_rows, oc_rows, ni_rows, nc_rows, vertex_positions)

    k8 = (jnp.zeros((8, T_OLD), jnp.float32)
          .at[0].set(kx).at[1].set(ky).at[2].set(kz))
    q8 = (jnp.zeros((T_NEW, 8), jnp.float32)
          .at[:, 0].set(qx).at[:, 1].set(qy).at[:, 2].set(qz))
    ksq2 = ksq.reshape(1, T_OLD)
    qsqb = jnp.broadcast_to(qsq[:, None], (T_NEW, 128))
    od2 = omel2.reshape(1, T_OLD)
    nd2 = nmel2.reshape(1, T_NEW)

    cands, weights, omel, nmel = _run_knn(q8, k8, ksq2, qsqb, od2, nd2)

    nmexp = jnp.broadcast_to(nmel.reshape(T_NEW, 1), (T_NEW, K))
    dens, = _run_dens(omel.reshape(T_OLD),
                      cands.reshape(T_NEW * K),
                      nmexp.reshape(T_NEW * K))
    return cands, weights, dens.reshape(T_NEW, K)


# speedup vs baseline: 4.6101x; 1.1322x over previous
import functools

import jax
import jax.numpy as jnp
from jax import lax
from jax.experimental import pallas as pl
from jax.experimental.pallas import tpu as pltpu
from jax.experimental.pallas import tpu_sc as plsc

T_NEW = 8192
T_OLD = 32768
V = 10000
K = 5

NW = 32
LANES = 16
W_OLD = T_OLD // NW
W_NEW = T_NEW // NW

Q_TILE = 256
N_QT = T_NEW // Q_TILE
KCH = 8192
N_KCH = T_OLD // KCH

_PAIRS = ((0, 1), (0, 2), (0, 3), (1, 2), (1, 3), (2, 3))


def _prep_body(hoi0, hoi1, hoi2, hoi3, hoc0, hoc1, hoc2,
               hni0, hni1, hni2, hni3, hnc0, hnc1, hnc2, vp,
               kx, ky, kz, ksq, omel2, qx, qy, qz, qsq, nmel2,
               vp_v,
               oi0, oi1, oi2, oi3, oc0, oc1, oc2,
               ni0, ni1, ni2, ni3, nc0, nc1, nc2,
               ko0, ko1, ko2, ksq_v, om_v,
               qo0, qo1, qo2, qsq_v, nm_v):
    wid = lax.axis_index("s") * 2 + lax.axis_index("c")
    oi_v = [oi0, oi1, oi2, oi3]
    ni_v = [ni0, ni1, ni2, ni3]
    occ_v = [oc0, oc1, oc2]
    ncc_v = [nc0, nc1, nc2]
    ko_v = [ko0, ko1, ko2]
    qo_v = [qo0, qo1, qo2]

    hoi = [hoi0, hoi1, hoi2, hoi3]
    hni = [hni0, hni1, hni2, hni3]
    hoc = [hoc0, hoc1, hoc2]
    hnc = [hnc0, hnc1, hnc2]

    pltpu.sync_copy(vp, vp_v)
    for r in range(4):
        pltpu.sync_copy(hoi[r].at[pl.ds(wid * W_OLD, W_OLD)], oi_v[r])
        pltpu.sync_copy(hni[r].at[pl.ds(wid * W_NEW, W_NEW)], ni_v[r])
    for c in range(3):
        pltpu.sync_copy(hoc[c].at[pl.ds(wid * W_OLD, W_OLD)], occ_v[c])
        pltpu.sync_copy(hnc[c].at[pl.ds(wid * W_NEW, W_NEW)], ncc_v[c])

    def make_loop(idx_v, cc_v, out_v, sq_v, mel_v, width):
        def body(j, _):
            s = pl.ds(j * LANES, LANES)
            addrs = [idx_v[r][s] * 3 for r in range(4)]
            coords = [[plsc.load_gather(vp_v, [a + c]) for a in addrs]
                      for c in range(3)]
            pts = []
            for c in range(3):
                x0, x1, x2, x3 = coords[c]
                cent = ((x0 + x2) + (x1 + x3)) * 0.25
                pts.append(0.5 * (cent + cc_v[c][s]))
            px, py, pz = pts
            sq_v[s] = (px * px + pz * pz) + py * py
            mel = None
            for (a, b) in _PAIRS:
                dx = coords[0][a] - coords[0][b]
                dy = coords[1][a] - coords[1][b]
                dz = coords[2][a] - coords[2][b]
                e2 = (dx * dx + dy * dy) + dz * dz
                mel = e2 if mel is None else jnp.minimum(mel, e2)
            mel_v[s] = mel
            for c in range(3):
                out_v[c][s] = pts[c]
            return 0

        lax.fori_loop(0, width // LANES, body, 0)

    make_loop(oi_v, occ_v, ko_v, ksq_v, om_v, W_OLD)
    make_loop(ni_v, ncc_v, qo_v, qsq_v, nm_v, W_NEW)

    hko = [kx, ky, kz]
    hqo = [qx, qy, qz]
    for c in range(3):
        pltpu.sync_copy(ko_v[c], hko[c].at[pl.ds(wid * W_OLD, W_OLD)])
        pltpu.sync_copy(qo_v[c], hqo[c].at[pl.ds(wid * W_NEW, W_NEW)])
    pltpu.sync_copy(ksq_v, ksq.at[pl.ds(wid * W_OLD, W_OLD)])
    pltpu.sync_copy(om_v, omel2.at[pl.ds(wid * W_OLD, W_OLD)])
    pltpu.sync_copy(qsq_v, qsq.at[pl.ds(wid * W_NEW, W_NEW)])
    pltpu.sync_copy(nm_v, nmel2.at[pl.ds(wid * W_NEW, W_NEW)])


def _run_prep(oi_rows, oc_rows, ni_rows, nc_rows, vp):
    f = pl.kernel(
        _prep_body,
        out_type=(
            [jax.ShapeDtypeStruct((T_OLD,), jnp.float32)] * 5
            + [jax.ShapeDtypeStruct((T_NEW,), jnp.float32)] * 5
        ),
        mesh=plsc.VectorSubcoreMesh(core_axis_name="c", subcore_axis_name="s"),
        compiler_params=pltpu.CompilerParams(needs_layout_passes=False),
        scratch_types=(
            [pltpu.VMEM((V * 3,), jnp.float32)]
            + [pltpu.VMEM((W_OLD,), jnp.int32)] * 4
            + [pltpu.VMEM((W_OLD,), jnp.float32)] * 3
            + [pltpu.VMEM((W_NEW,), jnp.int32)] * 4
            + [pltpu.VMEM((W_NEW,), jnp.float32)] * 3
            + [pltpu.VMEM((W_OLD,), jnp.float32)] * 5
            + [pltpu.VMEM((W_NEW,), jnp.float32)] * 5
        ),
    )
    return f(*oi_rows, *oc_rows, *ni_rows, *nc_rows, vp.reshape(V * 3))


_BIG_I = 2 ** 30


def _knn_body(q8, k8, ksq, qsqb, od2, nd2,
              cands_o, wts_o, omel_o, nmel_o):
    qb = q8[...].astype(jnp.bfloat16)
    qsq = qsqb[...][:, 0:1]

    def extract5(d2, iot):
        vs, is_ = [], []
        for _ in range(K):
            m = jnp.min(d2, axis=1, keepdims=True)
            mi = jnp.min(jnp.where(d2 == m, iot, _BIG_I),
                         axis=1, keepdims=True)
            d2 = jnp.where(iot == mi, jnp.inf, d2)
            vs.append(m)
            is_.append(mi)
        return jnp.concatenate(vs, axis=1), jnp.concatenate(is_, axis=1)

    def chunk(c, carry):
        bv, bi = carry
        kblk = k8[:, pl.ds(c * KCH, KCH)].astype(jnp.bfloat16)
        qk = jnp.dot(qb, kblk, preferred_element_type=jnp.float32)
        a = qsq + ksq[:, pl.ds(c * KCH, KCH)]
        d2 = a - qk * 2.0
        iot = lax.broadcasted_iota(jnp.int32, (Q_TILE, KCH), 1) + c * KCH
        cv, ci = extract5(d2, iot)
        av = jnp.concatenate([bv, cv], axis=1)
        ai = jnp.concatenate([bi, ci], axis=1)
        nv, ni = extract5(av, ai)
        return nv, ni

    bv0 = jnp.full((Q_TILE, K), jnp.inf, jnp.float32)
    bi0 = jnp.full((Q_TILE, K), _BIG_I, jnp.int32)
    bv, bi = lax.fori_loop(0, N_KCH, chunk, (bv0, bi0))

    cands_o[...] = bi
    dist = jnp.sqrt(jnp.maximum(bv, 1e-12))
    z = -dist
    zm = jnp.max(z, axis=1, keepdims=True)
    e = jnp.exp(z - zm)
    wts_o[...] = e / jnp.sum(e, axis=1, keepdims=True)
    omel_o[...] = jnp.sqrt(od2[...] + 1e-12)
    nmel_o[...] = jnp.sqrt(nd2[...] + 1e-12)


def _run_knn(q8, k8, ksq2, qsqb, od2, nd2):
    return pl.pallas_call(
        _knn_body,
        grid=(N_QT,),
        in_specs=[
            pl.BlockSpec((Q_TILE, 8), lambda i: (i, 0)),
            pl.BlockSpec((8, T_OLD), lambda i: (0, 0)),
            pl.BlockSpec((1, T_OLD), lambda i: (0, 0)),
            pl.BlockSpec((Q_TILE, 128), lambda i: (i, 0)),
            pl.BlockSpec((1, W_OLD), lambda i: (0, i)),
            pl.BlockSpec((1, W_NEW), lambda i: (0, i)),
        ],
        out_specs=[
            pl.BlockSpec((Q_TILE, K), lambda i: (i, 0)),
            pl.BlockSpec((Q_TILE, K), lambda i: (i, 0)),
            pl.BlockSpec((1, W_OLD), lambda i: (0, i)),
            pl.BlockSpec((1, W_NEW), lambda i: (0, i)),
        ],
        out_shape=[
            jax.ShapeDtypeStruct((T_NEW, K), jnp.int32),
            jax.ShapeDtypeStruct((T_NEW, K), jnp.float32),
            jax.ShapeDtypeStruct((1, T_OLD), jnp.float32),
            jax.ShapeDtypeStruct((1, T_NEW), jnp.float32),
        ],
    )(q8, k8, ksq2, qsqb, od2, nd2)


W_D = T_NEW * K // NW


def _dens_body(omel, cands, nmexp, dens, mel_v, idx_v, nm_v, out_v):
    wid = lax.axis_index("s") * 2 + lax.axis_index("c")
    base = wid * W_D
    pltpu.sync_copy(omel, mel_v)
    pltpu.sync_copy(cands.at[pl.ds(base, W_D)], idx_v)
    pltpu.sync_copy(nmexp.at[pl.ds(base, W_D)], nm_v)

    def body(j, _):
        s = pl.ds(j * LANES, LANES)
        g = plsc.load_gather(mel_v, [idx_v[s]])
        out_v[s] = nm_v[s] / (g + 1e-8)
        return 0

    lax.fori_loop(0, W_D // LANES, body, 0)
    pltpu.sync_copy(out_v, dens.at[pl.ds(base, W_D)])


def _run_dens(omel, cands_flat, nmexp_flat):
    f = pl.kernel(
        _dens_body,
        out_type=[jax.ShapeDtypeStruct((T_NEW * K,), jnp.float32)],
        mesh=plsc.VectorSubcoreMesh(core_axis_name="c", subcore_axis_name="s"),
        compiler_params=pltpu.CompilerParams(needs_layout_passes=False),
        scratch_types=[
            pltpu.VMEM((T_OLD,), jnp.float32),
            pltpu.VMEM((W_D,), jnp.int32),
            pltpu.VMEM((W_D,), jnp.float32),
            pltpu.VMEM((W_D,), jnp.float32),
        ],
    )
    return f(omel, cands_flat, nmexp_flat)


def kernel(new_indices, old_indices, old_cc, new_cc, vertex_positions):
    oi_rows = [old_indices[:, r] for r in range(4)]
    ni_rows = [new_indices[:, r] for r in range(4)]
    oc_rows = [old_cc[:, c] for c in range(3)]
    nc_rows = [new_cc[:, c] for c in range(3)]

    kx, ky, kz, ksq, omel2, qx, qy, qz, qsq, nmel2 = _run_prep(
        oi_rows, oc_rows, ni_rows, nc_rows, vertex_positions)

    k8 = (jnp.zeros((8, T_OLD), jnp.float32)
          .at[0].set(kx).at[1].set(ky).at[2].set(kz))
    q8 = (jnp.zeros((T_NEW, 8), jnp.float32)
          .at[:, 0].set(qx).at[:, 1].set(qy).at[:, 2].set(qz))
    ksq2 = ksq.reshape(1, T_OLD)
    qsqb = jnp.broadcast_to(qsq[:, None], (T_NEW, 128))
    od2 = omel2.reshape(1, T_OLD)
    nd2 = nmel2.reshape(1, T_NEW)

    cands, weights, omel, nmel = _run_knn(q8, k8, ksq2, qsqb, od2, nd2)

    nmexp = jnp.broadcast_to(nmel.reshape(T_NEW, 1), (T_NEW, K))
    dens, = _run_dens(omel.reshape(T_OLD),
                      cands.reshape(T_NEW * K),
                      nmexp.reshape(T_NEW * K))
    return cands, weights, dens.reshape(T_NEW, K)


# speedup vs baseline: 4.6122x; 1.0005x over previous
import functools

import jax
import jax.numpy as jnp
from jax import lax
from jax.experimental import pallas as pl
from jax.experimental.pallas import tpu as pltpu
from jax.experimental.pallas import tpu_sc as plsc

T_NEW = 8192
T_OLD = 32768
V = 10000
K = 5

NW = 32
LANES = 16
W_OLD = T_OLD // NW
W_NEW = T_NEW // NW

Q_TILE = 256
N_QT = T_NEW // Q_TILE
KCH = 8192
N_KCH = T_OLD // KCH

_PAIRS = ((0, 1), (0, 2), (0, 3), (1, 2), (1, 3), (2, 3))


def _prep_body(hoi0, hoi1, hoi2, hoi3, hoc0, hoc1, hoc2,
               hni0, hni1, hni2, hni3, hnc0, hnc1, hnc2, vp,
               kx, ky, kz, ksq, omel2, qx, qy, qz, qsq, nmel2,
               vp_v,
               oi0, oi1, oi2, oi3, oc0, oc1, oc2,
               ni0, ni1, ni2, ni3, nc0, nc1, nc2,
               ko0, ko1, ko2, ksq_v, om_v,
               qo0, qo1, qo2, qsq_v, nm_v):
    wid = lax.axis_index("s") * 2 + lax.axis_index("c")
    oi_v = [oi0, oi1, oi2, oi3]
    ni_v = [ni0, ni1, ni2, ni3]
    occ_v = [oc0, oc1, oc2]
    ncc_v = [nc0, nc1, nc2]
    ko_v = [ko0, ko1, ko2]
    qo_v = [qo0, qo1, qo2]

    hoi = [hoi0, hoi1, hoi2, hoi3]
    hni = [hni0, hni1, hni2, hni3]
    hoc = [hoc0, hoc1, hoc2]
    hnc = [hnc0, hnc1, hnc2]

    pltpu.sync_copy(vp, vp_v)
    for r in range(4):
        pltpu.sync_copy(hoi[r].at[pl.ds(wid * W_OLD, W_OLD)], oi_v[r])
        pltpu.sync_copy(hni[r].at[pl.ds(wid * W_NEW, W_NEW)], ni_v[r])
    for c in range(3):
        pltpu.sync_copy(hoc[c].at[pl.ds(wid * W_OLD, W_OLD)], occ_v[c])
        pltpu.sync_copy(hnc[c].at[pl.ds(wid * W_NEW, W_NEW)], ncc_v[c])

    def make_loop(idx_v, cc_v, out_v, sq_v, mel_v, width):
        def body(j, _):
            s = pl.ds(j * LANES, LANES)
            addrs = [idx_v[r][s] * 3 for r in range(4)]
            coords = [[plsc.load_gather(vp_v, [a + c]) for a in addrs]
                      for c in range(3)]
            pts = []
            for c in range(3):
                x0, x1, x2, x3 = coords[c]
                cent = ((x0 + x2) + (x1 + x3)) * 0.25
                pts.append(0.5 * (cent + cc_v[c][s]))
            px, py, pz = pts
            sq_v[s] = (px * px + pz * pz) + py * py
            mel = None
            for (a, b) in _PAIRS:
                dx = coords[0][a] - coords[0][b]
                dy = coords[1][a] - coords[1][b]
                dz = coords[2][a] - coords[2][b]
                e2 = (dx * dx + dy * dy) + dz * dz
                mel = e2 if mel is None else jnp.minimum(mel, e2)
            mel_v[s] = mel
            for c in range(3):
                out_v[c][s] = pts[c]
            return 0

        lax.fori_loop(0, width // LANES, body, 0)

    make_loop(oi_v, occ_v, ko_v, ksq_v, om_v, W_OLD)
    make_loop(ni_v, ncc_v, qo_v, qsq_v, nm_v, W_NEW)

    hko = [kx, ky, kz]
    hqo = [qx, qy, qz]
    for c in range(3):
        pltpu.sync_copy(ko_v[c], hko[c].at[pl.ds(wid * W_OLD, W_OLD)])
        pltpu.sync_copy(qo_v[c], hqo[c].at[pl.ds(wid * W_NEW, W_NEW)])
    pltpu.sync_copy(ksq_v, ksq.at[pl.ds(wid * W_OLD, W_OLD)])
    pltpu.sync_copy(om_v, omel2.at[pl.ds(wid * W_OLD, W_OLD)])
    pltpu.sync_copy(qsq_v, qsq.at[pl.ds(wid * W_NEW, W_NEW)])
    pltpu.sync_copy(nm_v, nmel2.at[pl.ds(wid * W_NEW, W_NEW)])


def _run_prep(oi_rows, oc_rows, ni_rows, nc_rows, vp):
    f = pl.kernel(
        _prep_body,
        out_type=(
            [jax.ShapeDtypeStruct((T_OLD,), jnp.float32)] * 5
            + [jax.ShapeDtypeStruct((T_NEW,), jnp.float32)] * 5
        ),
        mesh=plsc.VectorSubcoreMesh(core_axis_name="c", subcore_axis_name="s"),
        compiler_params=pltpu.CompilerParams(needs_layout_passes=False),
        scratch_types=(
            [pltpu.VMEM((V * 3,), jnp.float32)]
            + [pltpu.VMEM((W_OLD,), jnp.int32)] * 4
            + [pltpu.VMEM((W_OLD,), jnp.float32)] * 3
            + [pltpu.VMEM((W_NEW,), jnp.int32)] * 4
            + [pltpu.VMEM((W_NEW,), jnp.float32)] * 3
            + [pltpu.VMEM((W_OLD,), jnp.float32)] * 5
            + [pltpu.VMEM((W_NEW,), jnp.float32)] * 5
        ),
    )
    return f(*oi_rows, *oc_rows, *ni_rows, *nc_rows, vp.reshape(V * 3))


_BIG_I = 2 ** 30


def _knn_body(q8, k8, ksq, qsqb, od2, nd2,
              cands_o, wts_o, omel_o, nmel_o):
    qb = q8[...].astype(jnp.bfloat16)
    qsq = qsqb[...][:, 0:1]

    def extract5_chunk(d2, iot, ibase):
        vs, is_ = [], []
        for t in range(K):
            m = jnp.min(d2, axis=1, keepdims=True)
            mi = jnp.min(jnp.where(d2 == m, iot, _BIG_I),
                         axis=1, keepdims=True)
            if t < K - 1:
                d2 = jnp.where(iot == mi, jnp.inf, d2)
            vs.append(m)
            is_.append(mi + ibase)
        return jnp.concatenate(vs, axis=1), jnp.concatenate(is_, axis=1)

    def extract5_merge(av, ai):
        vs, is_ = [], []
        for t in range(K):
            m = jnp.min(av, axis=1, keepdims=True)
            mi = jnp.min(jnp.where(av == m, ai, _BIG_I),
                         axis=1, keepdims=True)
            if t < K - 1:
                av = jnp.where((av == m) & (ai == mi), jnp.inf, av)
            vs.append(m)
            is_.append(mi)
        return jnp.concatenate(vs, axis=1), jnp.concatenate(is_, axis=1)

    def chunk(c, carry):
        bv, bi = carry
        kblk = k8[:, pl.ds(c * KCH, KCH)].astype(jnp.bfloat16)
        qk = jnp.dot(qb, kblk, preferred_element_type=jnp.float32)
        a = qsq + ksq[:, pl.ds(c * KCH, KCH)]
        d2 = a - qk * 2.0
        iot = lax.broadcasted_iota(jnp.int32, (Q_TILE, KCH), 1)
        cv, ci = extract5_chunk(d2, iot, c * KCH)
        av = jnp.concatenate([bv, cv], axis=1)
        ai = jnp.concatenate([bi, ci], axis=1)
        return extract5_merge(av, ai)

    bv0 = jnp.full((Q_TILE, K), jnp.inf, jnp.float32)
    bi0 = jnp.full((Q_TILE, K), _BIG_I, jnp.int32)
    bv, bi = lax.fori_loop(0, N_KCH, chunk, (bv0, bi0))

    cands_o[...] = bi
    dist = jnp.sqrt(jnp.maximum(bv, 1e-12))
    z = -dist
    zm = jnp.max(z, axis=1, keepdims=True)
    e = jnp.exp(z - zm)
    wts_o[...] = e / jnp.sum(e, axis=1, keepdims=True)
    omel_o[...] = jnp.sqrt(od2[...] + 1e-12)
    nmel_o[...] = jnp.sqrt(nd2[...] + 1e-12)


def _run_knn(q8, k8, ksq2, qsqb, od2, nd2):
    return pl.pallas_call(
        _knn_body,
        grid=(N_QT,),
        in_specs=[
            pl.BlockSpec((Q_TILE, 8), lambda i: (i, 0)),
            pl.BlockSpec((8, T_OLD), lambda i: (0, 0)),
            pl.BlockSpec((1, T_OLD), lambda i: (0, 0)),
            pl.BlockSpec((Q_TILE, 128), lambda i: (i, 0)),
            pl.BlockSpec((1, W_OLD), lambda i: (0, i)),
            pl.BlockSpec((1, W_NEW), lambda i: (0, i)),
        ],
        out_specs=[
            pl.BlockSpec((Q_TILE, K), lambda i: (i, 0)),
            pl.BlockSpec((Q_TILE, K), lambda i: (i, 0)),
            pl.BlockSpec((1, W_OLD), lambda i: (0, i)),
            pl.BlockSpec((1, W_NEW), lambda i: (0, i)),
        ],
        out_shape=[
            jax.ShapeDtypeStruct((T_NEW, K), jnp.int32),
            jax.ShapeDtypeStruct((T_NEW, K), jnp.float32),
            jax.ShapeDtypeStruct((1, T_OLD), jnp.float32),
            jax.ShapeDtypeStruct((1, T_NEW), jnp.float32),
        ],
    )(q8, k8, ksq2, qsqb, od2, nd2)


W_D = T_NEW * K // NW


def _dens_body(omel, cands, nmexp, dens, mel_v, idx_v, nm_v, out_v):
    wid = lax.axis_index("s") * 2 + lax.axis_index("c")
    base = wid * W_D
    pltpu.sync_copy(omel, mel_v)
    pltpu.sync_copy(cands.at[pl.ds(base, W_D)], idx_v)
    pltpu.sync_copy(nmexp.at[pl.ds(base, W_D)], nm_v)

    def body(j, _):
        s = pl.ds(j * LANES, LANES)
        g = plsc.load_gather(mel_v, [idx_v[s]])
        out_v[s] = nm_v[s] / (g + 1e-8)
        return 0

    lax.fori_loop(0, W_D // LANES, body, 0)
    pltpu.sync_copy(out_v, dens.at[pl.ds(base, W_D)])


def _run_dens(omel, cands_flat, nmexp_flat):
    f = pl.kernel(
        _dens_body,
        out_type=[jax.ShapeDtypeStruct((T_NEW * K,), jnp.float32)],
        mesh=plsc.VectorSubcoreMesh(core_axis_name="c", subcore_axis_name="s"),
        compiler_params=pltpu.CompilerParams(needs_layout_passes=False),
        scratch_types=[
            pltpu.VMEM((T_OLD,), jnp.float32),
            pltpu.VMEM((W_D,), jnp.int32),
            pltpu.VMEM((W_D,), jnp.float32),
            pltpu.VMEM((W_D,), jnp.float32),
        ],
    )
    return f(omel, cands_flat, nmexp_flat)


def kernel(new_indices, old_indices, old_cc, new_cc, vertex_positions):
    oi_rows = [old_indices[:, r] for r in range(4)]
    ni_rows = [new_indices[:, r] for r in range(4)]
    oc_rows = [old_cc[:, c] for c in range(3)]
    nc_rows = [new_cc[:, c] for c in range(3)]

    kx, ky, kz, ksq, omel2, qx, qy, qz, qsq, nmel2 = _run_prep(
        oi_rows, oc_rows, ni_rows, nc_rows, vertex_positions)

    k8 = (jnp.zeros((8, T_OLD), jnp.float32)
          .at[0].set(kx).at[1].set(ky).at[2].set(kz))
    q8 = (jnp.zeros((T_NEW, 8), jnp.float32)
          .at[:, 0].set(qx).at[:, 1].set(qy).at[:, 2].set(qz))
    ksq2 = ksq.reshape(1, T_OLD)
    qsqb = jnp.broadcast_to(qsq[:, None], (T_NEW, 128))
    od2 = omel2.reshape(1, T_OLD)
    nd2 = nmel2.reshape(1, T_NEW)

    cands, weights, omel, nmel = _run_knn(q8, k8, ksq2, qsqb, od2, nd2)

    nmexp = jnp.broadcast_to(nmel.reshape(T_NEW, 1), (T_NEW, K))
    dens, = _run_dens(omel.reshape(T_OLD),
                      cands.reshape(T_NEW * K),
                      nmexp.reshape(T_NEW * K))
    return cands, weights, dens.reshape(T_NEW, K)
